# baseline (speedup 1.0000x reference)
import functools

import jax
import jax.numpy as jnp
from jax import lax
from jax.experimental import pallas as pl
from jax.experimental.pallas import tpu as pltpu
from jax.experimental.pallas import tpu_sc as plsc

D = 128
K = 16
NC = 2
NS = 16
NW = NC * NS
CH = 8
SCH = 64


def _sc_gather_sum(features, nodes_p, n0_p, n1_p, bp):
    npw = bp // NW
    n_chunks = npw // CH
    s_chunks = npw // SCH
    mesh = plsc.VectorSubcoreMesh(core_axis_name="c", subcore_axis_name="s")
    f32 = jnp.float32

    @functools.partial(
        pl.kernel,
        out_type=(
            jax.ShapeDtypeStruct((bp, D), f32),
            jax.ShapeDtypeStruct((bp, D), f32),
            jax.ShapeDtypeStruct((bp, D), f32),
        ),
        mesh=mesh,
        scratch_types=[
            pltpu.VMEM((SCH,), jnp.int32),
            pltpu.VMEM((SCH, D), f32),
            pltpu.VMEM((CH * K,), jnp.int32),
            pltpu.VMEM((CH * K, D), f32),
            pltpu.VMEM((CH, D), f32),
            pltpu.SemaphoreType.DMA,
        ],
    )
    def sc_kernel(feat_hbm, nodes_hbm, n0_hbm, n1_hbm,
                  s_hbm, n0s_hbm, n1s_hbm,
                  sidx_v, srows_v, idx_v, rows_v, out_v, sem):
        wid = lax.axis_index("s") * NC + lax.axis_index("c")
        nbase = wid * npw

        @pl.loop(0, s_chunks)
        def _(ci):
            b = nbase + ci * SCH
            pltpu.sync_copy(nodes_hbm.at[pl.ds(b, SCH)], sidx_v)
            pltpu.async_copy(feat_hbm.at[sidx_v], srows_v, sem).wait()
            pltpu.sync_copy(srows_v, s_hbm.at[pl.ds(b, SCH)])

        def one_table(tbl_hbm, out_hbm):
            @pl.loop(0, n_chunks)
            def _(ci):
                nb = nbase + ci * CH
                pltpu.sync_copy(tbl_hbm.at[pl.ds(nb * K, CH * K)], idx_v)
                pltpu.async_copy(feat_hbm.at[idx_v], rows_v, sem).wait()

                @pl.loop(0, CH)
                def _(i):
                    base = i * K

                    def body(k, accs):
                        return tuple(
                            accs[j] + rows_v[base + k, pl.ds(j * 16, 16)]
                            for j in range(D // 16)
                        )

                    accs = lax.fori_loop(
                        0, K, body,
                        tuple(jnp.zeros((16,), f32) for _ in range(D // 16)))
                    for j in range(D // 16):
                        out_v[i, pl.ds(j * 16, 16)] = accs[j]

                pltpu.sync_copy(out_v, out_hbm.at[pl.ds(nb, CH)])

        one_table(n0_hbm, n0s_hbm)
        one_table(n1_hbm, n1s_hbm)

    return sc_kernel(features, nodes_p, n0_p, n1_p)


def _tc_matmul(weight, s, n0s, n1s):
    bp = s.shape[0]
    blk = 512
    dn = (((1,), (1,)), ((), ()))

    def body(w_ref, s_ref, n0_ref, n1_ref, o_ref):
        w = w_ref[...]
        acc = lax.dot_general(w[:, 0:D], s_ref[...], dn,
                              preferred_element_type=jnp.float32)
        wn = w[:, D:3 * D] * jnp.float32(1.0 / K)
        acc = acc + lax.dot_general(wn[:, 0:D], n0_ref[...], dn,
                                    preferred_element_type=jnp.float32)
        acc = acc + lax.dot_general(wn[:, D:2 * D], n1_ref[...], dn,
                                    preferred_element_type=jnp.float32)
        o_ref[...] = jnp.maximum(acc, 0.0)

    return pl.pallas_call(
        body,
        grid=(bp // blk,),
        in_specs=[
            pl.BlockSpec((D, 3 * D), lambda i: (0, 0)),
            pl.BlockSpec((blk, D), lambda i: (i, 0)),
            pl.BlockSpec((blk, D), lambda i: (i, 0)),
            pl.BlockSpec((blk, D), lambda i: (i, 0)),
        ],
        out_specs=pl.BlockSpec((D, blk), lambda i: (0, i)),
        out_shape=jax.ShapeDtypeStruct((D, bp), jnp.float32),
    )(weight, s, n0s, n1s)


def kernel(nodes, neigh0, neigh1, features, weight):
    b = nodes.shape[0]
    bp = -(-b // (SCH * NW)) * (SCH * NW)
    pad = bp - b
    nodes_p = jnp.concatenate(
        [nodes.astype(jnp.int32), jnp.zeros((pad,), jnp.int32)])
    n0_p = jnp.concatenate(
        [neigh0.astype(jnp.int32).reshape(-1), jnp.zeros((pad * K,), jnp.int32)])
    n1_p = jnp.concatenate(
        [neigh1.astype(jnp.int32).reshape(-1), jnp.zeros((pad * K,), jnp.int32)])
    s, n0s, n1s = _sc_gather_sum(features, nodes_p, n0_p, n1_p, bp)
    out = _tc_matmul(weight, s, n0s, n1s)
    return out[:, :b]


# speedup vs baseline: 1.1506x; 1.1506x over previous
import functools

import jax
import jax.numpy as jnp
from jax import lax
from jax.experimental import pallas as pl
from jax.experimental.pallas import tpu as pltpu
from jax.experimental.pallas import tpu_sc as plsc

D = 128
K = 16
NC = 2
NS = 16
NW = NC * NS
CH = 8
CHK = CH * K
SCH = 64


def _sc_gather_sum(features, nodes_p, n0_p, n1_p, bp):
    npw = bp // NW
    n_chunks = npw // CH
    s_chunks = npw // SCH
    mesh = plsc.VectorSubcoreMesh(core_axis_name="c", subcore_axis_name="s")
    f32 = jnp.float32

    @functools.partial(
        pl.kernel,
        out_type=(
            jax.ShapeDtypeStruct((bp, D), f32),
            jax.ShapeDtypeStruct((bp, D), f32),
            jax.ShapeDtypeStruct((bp, D), f32),
        ),
        mesh=mesh,
        scratch_types=[
            pltpu.VMEM((s_chunks, SCH), jnp.int32),
            pltpu.VMEM((npw, D), f32),
            pltpu.VMEM((CHK,), jnp.int32),
            pltpu.VMEM((CHK,), jnp.int32),
            pltpu.VMEM((CHK, D), f32),
            pltpu.VMEM((CHK, D), f32),
            pltpu.VMEM((CH, D), f32),
            pltpu.SemaphoreType.DMA,
            pltpu.SemaphoreType.DMA,
            pltpu.SemaphoreType.DMA,
            pltpu.SemaphoreType.DMA,
            pltpu.SemaphoreType.DMA,
        ],
    )
    def sc_kernel(feat_hbm, nodes_hbm, n0_hbm, n1_hbm,
                  s_hbm, n0s_hbm, n1s_hbm,
                  sidx_v, srows_v, idx_a, idx_b, rows_a, rows_b, out_v,
                  sem_a, sem_b, sem_ia, sem_ib, sem_s):
        wid = lax.axis_index("s") * NC + lax.axis_index("c")
        nbase = wid * npw

        for si in range(s_chunks):
            pltpu.sync_copy(nodes_hbm.at[pl.ds(nbase + si * SCH, SCH)],
                            sidx_v.at[si])
        for si in range(s_chunks):
            pltpu.async_copy(feat_hbm.at[sidx_v.at[si]],
                             srows_v.at[pl.ds(si * SCH, SCH)], sem_s)

        def accumulate(rows_v):
            @pl.loop(0, CH)
            def _(i):
                base = i * K
                for j in range(D // 16):
                    acc = rows_v[base, pl.ds(j * 16, 16)]
                    for k in range(1, K):
                        acc = acc + rows_v[base + k, pl.ds(j * 16, 16)]
                    out_v[i, pl.ds(j * 16, 16)] = acc

        def one_table(tbl_hbm, out_hbm):
            ibase = nbase * K

            def half(idx_v, rows_v, sem_i, sem_g, ci, ci_next):
                pltpu.make_async_copy(feat_hbm.at[idx_v], rows_v,
                                      sem_g).wait()

                @pl.when(ci_next < n_chunks)
                def _():
                    pltpu.async_copy(
                        tbl_hbm.at[pl.ds(ibase + ci_next * CHK, CHK)],
                        idx_v, sem_i)

                accumulate(rows_v)
                pltpu.sync_copy(out_v, out_hbm.at[pl.ds(nbase + ci * CH, CH)])

                @pl.when(ci_next < n_chunks)
                def _():
                    pltpu.make_async_copy(
                        tbl_hbm.at[pl.ds(ibase, CHK)], idx_v, sem_i).wait()
                    pltpu.async_copy(feat_hbm.at[idx_v], rows_v, sem_g)

            pltpu.sync_copy(tbl_hbm.at[pl.ds(ibase, CHK)], idx_a)
            pltpu.async_copy(feat_hbm.at[idx_a], rows_a, sem_a)
            pltpu.sync_copy(tbl_hbm.at[pl.ds(ibase + CHK, CHK)], idx_b)
            pltpu.async_copy(feat_hbm.at[idx_b], rows_b, sem_b)

            @pl.loop(0, n_chunks, step=2)
            def _(ci):
                half(idx_a, rows_a, sem_ia, sem_a, ci, ci + 2)
                half(idx_b, rows_b, sem_ib, sem_b, ci + 1, ci + 3)

        one_table(n0_hbm, n0s_hbm)
        for si in range(s_chunks):
            pltpu.make_async_copy(feat_hbm.at[sidx_v.at[si]],
                                  srows_v.at[pl.ds(si * SCH, SCH)],
                                  sem_s).wait()
        pltpu.sync_copy(srows_v, s_hbm.at[pl.ds(nbase, npw)])
        one_table(n1_hbm, n1s_hbm)

    return sc_kernel(features, nodes_p, n0_p, n1_p)


def _tc_matmul(weight, s, n0s, n1s):
    bp = s.shape[0]
    blk = 512
    dn = (((1,), (1,)), ((), ()))

    def body(w_ref, s_ref, n0_ref, n1_ref, o_ref):
        w = w_ref[...]
        acc = lax.dot_general(w[:, 0:D], s_ref[...], dn,
                              preferred_element_type=jnp.float32)
        wn = w[:, D:3 * D] * jnp.float32(1.0 / K)
        acc = acc + lax.dot_general(wn[:, 0:D], n0_ref[...], dn,
                                    preferred_element_type=jnp.float32)
        acc = acc + lax.dot_general(wn[:, D:2 * D], n1_ref[...], dn,
                                    preferred_element_type=jnp.float32)
        o_ref[...] = jnp.maximum(acc, 0.0)

    return pl.pallas_call(
        body,
        grid=(bp // blk,),
        in_specs=[
            pl.BlockSpec((D, 3 * D), lambda i: (0, 0)),
            pl.BlockSpec((blk, D), lambda i: (i, 0)),
            pl.BlockSpec((blk, D), lambda i: (i, 0)),
            pl.BlockSpec((blk, D), lambda i: (i, 0)),
        ],
        out_specs=pl.BlockSpec((D, blk), lambda i: (0, i)),
        out_shape=jax.ShapeDtypeStruct((D, bp), jnp.float32),
    )(weight, s, n0s, n1s)


def kernel(nodes, neigh0, neigh1, features, weight):
    b = nodes.shape[0]
    bp = -(-b // (SCH * NW)) * (SCH * NW)
    pad = bp - b
    nodes_p = jnp.concatenate(
        [nodes.astype(jnp.int32), jnp.zeros((pad,), jnp.int32)])
    n0_p = jnp.concatenate(
        [neigh0.astype(jnp.int32).reshape(-1), jnp.zeros((pad * K,), jnp.int32)])
    n1_p = jnp.concatenate(
        [neigh1.astype(jnp.int32).reshape(-1), jnp.zeros((pad * K,), jnp.int32)])
    s, n0s, n1s = _sc_gather_sum(features, nodes_p, n0_p, n1_p, bp)
    out = _tc_matmul(weight, s, n0s, n1s)
    return out[:, :b]


# speedup vs baseline: 1.1725x; 1.0191x over previous
import functools

import jax
import jax.numpy as jnp
from jax import lax
from jax.experimental import pallas as pl
from jax.experimental.pallas import tpu as pltpu
from jax.experimental.pallas import tpu_sc as plsc

D = 128
K = 16
NC = 2
NS = 16
NW = NC * NS
CH = 8
CHK = CH * K
SCH = 64
ZR = 64


def _sc_gather_sum(features, nodes_p, n0_p, n1_p, bp):
    npw = bp // NW
    n_chunks = npw // CH
    s_chunks = npw // SCH
    z_copies = npw // ZR
    mesh = plsc.VectorSubcoreMesh(core_axis_name="c", subcore_axis_name="s")
    f32 = jnp.float32

    @functools.partial(
        pl.kernel,
        out_type=(
            jax.ShapeDtypeStruct((bp, D), f32),
            jax.ShapeDtypeStruct((bp, D), f32),
            jax.ShapeDtypeStruct((bp, D), f32),
        ),
        mesh=mesh,
        scratch_types=[
            pltpu.VMEM((s_chunks, SCH), jnp.int32),
            pltpu.VMEM((npw, D), f32),
            pltpu.VMEM((CHK,), jnp.int32),
            pltpu.VMEM((CHK,), jnp.int32),
            pltpu.VMEM((CHK, D), f32),
            pltpu.VMEM((CHK, D), f32),
            pltpu.VMEM((ZR, D), f32),
            pltpu.VMEM((CHK,), jnp.int32),
            pltpu.VMEM_SHARED((NS * npw, D), f32),
            pltpu.SemaphoreType.DMA,
            pltpu.SemaphoreType.DMA,
            pltpu.SemaphoreType.DMA,
            pltpu.SemaphoreType.DMA,
            pltpu.SemaphoreType.DMA,
            pltpu.SemaphoreType.DMA,
        ],
    )
    def sc_kernel(feat_hbm, nodes_hbm, n0_hbm, n1_hbm,
                  s_hbm, n0s_hbm, n1s_hbm,
                  sidx_v, srows_v, idx_a, idx_b, rows_a, rows_b,
                  zer_v, aidx_v, acc_sh,
                  sem_a, sem_b, sem_ia, sem_ib, sem_s, sem_o):
        sid = lax.axis_index("s")
        wid = sid * NC + lax.axis_index("c")
        nbase = wid * npw
        abase = sid * npw

        for r in range(ZR):
            for j in range(D // 16):
                zer_v[r, pl.ds(j * 16, 16)] = jnp.zeros((16,), f32)
        for zi in range(z_copies):
            pltpu.sync_copy(zer_v, acc_sh.at[pl.ds(abase + zi * ZR, ZR)])

        for si in range(s_chunks):
            pltpu.sync_copy(nodes_hbm.at[pl.ds(nbase + si * SCH, SCH)],
                            sidx_v.at[si])
        for si in range(s_chunks):
            pltpu.async_copy(feat_hbm.at[sidx_v.at[si]],
                             srows_v.at[pl.ds(si * SCH, SCH)], sem_s)

        def one_table(tbl_hbm, out_hbm):
            ibase = nbase * K

            def half(idx_v, rows_v, sem_i, sem_g, ci, ci_next):
                pltpu.make_async_copy(feat_hbm.at[idx_v], rows_v,
                                      sem_g).wait()

                @pl.when(ci_next < n_chunks)
                def _():
                    pltpu.async_copy(
                        tbl_hbm.at[pl.ds(ibase + ci_next * CHK, CHK)],
                        idx_v, sem_i)

                for j in range(CH):
                    aidx_v[pl.ds(j * K, K)] = lax.full(
                        (16,), abase + ci * CH + j, jnp.int32)
                pltpu.sync_copy(rows_v, acc_sh.at[aidx_v], add=True)

                @pl.when(ci_next < n_chunks)
                def _():
                    pltpu.make_async_copy(
                        tbl_hbm.at[pl.ds(ibase, CHK)], idx_v, sem_i).wait()
                    pltpu.async_copy(feat_hbm.at[idx_v], rows_v, sem_g)

            pltpu.sync_copy(tbl_hbm.at[pl.ds(ibase, CHK)], idx_a)
            pltpu.async_copy(feat_hbm.at[idx_a], rows_a, sem_a)
            pltpu.sync_copy(tbl_hbm.at[pl.ds(ibase + CHK, CHK)], idx_b)
            pltpu.async_copy(feat_hbm.at[idx_b], rows_b, sem_b)

            @pl.loop(0, n_chunks, step=2)
            def _(ci):
                half(idx_a, rows_a, sem_ia, sem_a, ci, ci + 2)
                half(idx_b, rows_b, sem_ib, sem_b, ci + 1, ci + 3)

            pltpu.async_copy(acc_sh.at[pl.ds(abase, npw)],
                             out_hbm.at[pl.ds(nbase, npw)], sem_o)

        one_table(n0_hbm, n0s_hbm)
        for si in range(s_chunks):
            pltpu.make_async_copy(feat_hbm.at[sidx_v.at[si]],
                                  srows_v.at[pl.ds(si * SCH, SCH)],
                                  sem_s).wait()
        pltpu.sync_copy(srows_v, s_hbm.at[pl.ds(nbase, npw)])
        pltpu.make_async_copy(acc_sh.at[pl.ds(abase, npw)],
                              n0s_hbm.at[pl.ds(nbase, npw)], sem_o).wait()
        for zi in range(z_copies):
            pltpu.sync_copy(zer_v, acc_sh.at[pl.ds(abase + zi * ZR, ZR)])
        one_table(n1_hbm, n1s_hbm)
        pltpu.make_async_copy(acc_sh.at[pl.ds(abase, npw)],
                              n1s_hbm.at[pl.ds(nbase, npw)], sem_o).wait()

    return sc_kernel(features, nodes_p, n0_p, n1_p)


def _tc_matmul(weight, s, n0s, n1s):
    bp = s.shape[0]
    blk = 512
    dn = (((1,), (1,)), ((), ()))

    def body(w_ref, s_ref, n0_ref, n1_ref, o_ref):
        w = w_ref[...]
        acc = lax.dot_general(w[:, 0:D], s_ref[...], dn,
                              preferred_element_type=jnp.float32)
        wn = w[:, D:3 * D] * jnp.float32(1.0 / K)
        acc = acc + lax.dot_general(wn[:, 0:D], n0_ref[...], dn,
                                    preferred_element_type=jnp.float32)
        acc = acc + lax.dot_general(wn[:, D:2 * D], n1_ref[...], dn,
                                    preferred_element_type=jnp.float32)
        o_ref[...] = jnp.maximum(acc, 0.0)

    return pl.pallas_call(
        body,
        grid=(bp // blk,),
        in_specs=[
            pl.BlockSpec((D, 3 * D), lambda i: (0, 0)),
            pl.BlockSpec((blk, D), lambda i: (i, 0)),
            pl.BlockSpec((blk, D), lambda i: (i, 0)),
            pl.BlockSpec((blk, D), lambda i: (i, 0)),
        ],
        out_specs=pl.BlockSpec((D, blk), lambda i: (0, i)),
        out_shape=jax.ShapeDtypeStruct((D, bp), jnp.float32),
    )(weight, s, n0s, n1s)


def kernel(nodes, neigh0, neigh1, features, weight):
    b = nodes.shape[0]
    bp = -(-b // (SCH * NW)) * (SCH * NW)
    pad = bp - b
    nodes_p = jnp.concatenate(
        [nodes.astype(jnp.int32), jnp.zeros((pad,), jnp.int32)])
    n0_p = jnp.concatenate(
        [neigh0.astype(jnp.int32).reshape(-1), jnp.zeros((pad * K,), jnp.int32)])
    n1_p = jnp.concatenate(
        [neigh1.astype(jnp.int32).reshape(-1), jnp.zeros((pad * K,), jnp.int32)])
    s, n0s, n1s = _sc_gather_sum(features, nodes_p, n0_p, n1_p, bp)
    out = _tc_matmul(weight, s, n0s, n1s)
    return out[:, :b]
